# baseline (speedup 1.0000x reference)
import functools

import jax
import jax.numpy as jnp
from jax import lax
from jax.experimental import pallas as pl
from jax.experimental.pallas import tpu as pltpu
from jax.experimental.pallas import tpu_sc as plsc

LOWER = -20.0
UPPER = 20.0
NUM_BINS = 255
BIN_WIDTH = (UPPER - LOWER) / (NUM_BINS - 1)
INV_W = 1.0 / BIN_WIDTH

L = 16
C = 128
GROUPS = C // L
NCHUNK = 128


def _sc_body(x_hbm, out_hbm, xb, slab0, slab1, slab2, idx0, idx1, idx2,
             sem0, sem1, sem2):
    wid = lax.axis_index("s") * 2 + lax.axis_index("c")
    a = wid // 2
    h = wid % 2
    row0 = 8 * a
    col0 = 2048 * h

    for r in range(8):
        pltpu.sync_copy(
            x_hbm.at[row0 + r, pl.ds(col0, 2048)],
            xb.at[pl.ds(r * 2048, 2048)],
        )

    zeros16 = jnp.zeros((L,), jnp.float32)
    zidx16 = jnp.zeros((L,), jnp.int32)
    lane = lax.broadcasted_iota(jnp.int32, (L,), 0)

    def memset_body(c, _):
        for u in range(GROUPS):
            slab0[c, 0, pl.ds(u * L, L)] = zeros16
            slab1[c, 0, pl.ds(u * L, L)] = zeros16
            slab2[c, 0, pl.ds(u * L, L)] = zeros16
        return 0

    lax.fori_loop(0, NUM_BINS, memset_body, 0)

    def fill_chunk(k, slab, idxbuf):
        for g in range(GROUPS):
            xvec = xb[pl.ds(k * C + g * L, L)]
            t = (xvec - LOWER) * INV_W
            it = t.astype(jnp.int32)
            itf = it.astype(jnp.float32)
            idx = jnp.where(itf > t, it - 1, it)
            cl0 = jnp.minimum(jnp.maximum(idx, 0), NUM_BINS - 1)
            cl1 = jnp.minimum(jnp.maximum(idx + 1, 0), NUM_BINS - 1)
            cl0f = cl0.astype(jnp.float32)
            low_v = jnp.abs(LOWER + BIN_WIDTH + cl0f * BIN_WIDTH - xvec) * INV_W
            up_v = jnp.abs(LOWER + cl0f * BIN_WIDTH - xvec) * INV_W
            m0 = idx == cl0
            m1 = (idx + 1) == cl1
            pos = g * L + lane
            plsc.store_scatter(slab, [cl0, zidx16, pos], low_v, mask=m0)
            plsc.store_scatter(slab, [cl1, zidx16, pos], up_v, mask=m1)
            idxbuf[pl.ds(g * L, L)] = cl0
            idxbuf[pl.ds(C + g * L, L)] = cl1

    def clear_chunk(slab, idxbuf):
        for g in range(GROUPS):
            pos = g * L + lane
            c0 = idxbuf[pl.ds(g * L, L)]
            c1 = idxbuf[pl.ds(C + g * L, L)]
            plsc.store_scatter(slab, [c0, zidx16, pos], zeros16)
            plsc.store_scatter(slab, [c1, zidx16, pos], zeros16)

    def dst_for(k):
        i = row0 + (k >> 4)
        j = col0 + (k & 15) * C
        return out_hbm.at[:, pl.ds(i, 1), pl.ds(j, C)]

    slabs = (slab0, slab1, slab2)
    idxs = (idx0, idx1, idx2)
    sems = (sem0, sem1, sem2)

    for p in range(3):
        fill_chunk(jnp.int32(p), slabs[p], idxs[p])
        pltpu.make_async_copy(slabs[p], dst_for(jnp.int32(p)), sems[p]).start()

    def loop_body(m, _):
        for p in range(3):
            k = 3 * m + p
            pltpu.make_async_copy(slabs[p], dst_for(k - 3), sems[p]).wait()
            clear_chunk(slabs[p], idxs[p])
            fill_chunk(k, slabs[p], idxs[p])
            pltpu.make_async_copy(slabs[p], dst_for(k), sems[p]).start()
        return 0

    lax.fori_loop(1, 42, loop_body, 0)

    for p in range(2):
        k = jnp.int32(126 + p)
        pltpu.make_async_copy(slabs[p], dst_for(k - 3), sems[p]).wait()
        clear_chunk(slabs[p], idxs[p])
        fill_chunk(k, slabs[p], idxs[p])
        pltpu.make_async_copy(slabs[p], dst_for(k), sems[p]).start()

    for p in range(3):
        pltpu.make_async_copy(slabs[p], dst_for(jnp.int32(0)), sems[p]).wait()


def kernel(x):
    orig_shape = x.shape[:-1]
    b0, b1 = orig_shape
    xf = x.reshape(b0, b1)
    mesh = plsc.VectorSubcoreMesh(core_axis_name="c", subcore_axis_name="s")
    f = functools.partial(
        pl.kernel,
        mesh=mesh,
        out_type=jax.ShapeDtypeStruct((NUM_BINS, b0, b1), jnp.float32),
        scratch_types=[
            pltpu.VMEM((b0 * b1 // 32,), jnp.float32),
            pltpu.VMEM((NUM_BINS, 1, C), jnp.float32),
            pltpu.VMEM((NUM_BINS, 1, C), jnp.float32),
            pltpu.VMEM((NUM_BINS, 1, C), jnp.float32),
            pltpu.VMEM((2 * C,), jnp.int32),
            pltpu.VMEM((2 * C,), jnp.int32),
            pltpu.VMEM((2 * C,), jnp.int32),
            pltpu.SemaphoreType.DMA,
            pltpu.SemaphoreType.DMA,
            pltpu.SemaphoreType.DMA,
        ],
        compiler_params=pltpu.CompilerParams(
            needs_layout_passes=False, use_tc_tiling_on_sc=True
        ),
    )(_sc_body)
    out = f(xf)
    return out.transpose(1, 2, 0)


# speedup vs baseline: 1.0221x; 1.0221x over previous
import functools

import jax
import jax.numpy as jnp
from jax import lax
from jax.experimental import pallas as pl
from jax.experimental.pallas import tpu as pltpu
from jax.experimental.pallas import tpu_sc as plsc

LOWER = -20.0
UPPER = 20.0
NUM_BINS = 255
BIN_WIDTH = (UPPER - LOWER) / (NUM_BINS - 1)
INV_W = 1.0 / BIN_WIDTH

L = 16
C = 128
GROUPS = C // L
NCHUNK = 128


def _sc_body(x_hbm, out_hbm, xb, slab0, slab1, idx0, idx1, sem0, sem1):
    wid = lax.axis_index("s") * 2 + lax.axis_index("c")
    a = wid // 2
    h = wid % 2
    row0 = 8 * a
    col0 = 2048 * h

    for r in range(8):
        pltpu.sync_copy(
            x_hbm.at[row0 + r, pl.ds(col0, 2048)],
            xb.at[pl.ds(r * 2048, 2048)],
        )

    zeros16 = jnp.zeros((L,), jnp.float32)
    zidx16 = jnp.zeros((L,), jnp.int32)
    lane = lax.broadcasted_iota(jnp.int32, (L,), 0)

    def memset_body(c, _):
        for u in range(GROUPS):
            slab0[c, 0, pl.ds(u * L, L)] = zeros16
            slab1[c, 0, pl.ds(u * L, L)] = zeros16
        return 0

    lax.fori_loop(0, NUM_BINS, memset_body, 0)

    def fill_chunk(k, slab, idxbuf):
        for g in range(GROUPS):
            xvec = xb[pl.ds(k * C + g * L, L)]
            t = (xvec - LOWER) * INV_W
            it = t.astype(jnp.int32)
            itf = it.astype(jnp.float32)
            idx = jnp.where(itf > t, it - 1, it)
            cl0 = jnp.minimum(jnp.maximum(idx, 0), NUM_BINS - 1)
            cl1 = jnp.minimum(jnp.maximum(idx + 1, 0), NUM_BINS - 1)
            cl0f = cl0.astype(jnp.float32)
            low_v = jnp.abs(LOWER + BIN_WIDTH + cl0f * BIN_WIDTH - xvec) * INV_W
            up_v = jnp.abs(LOWER + cl0f * BIN_WIDTH - xvec) * INV_W
            m0 = idx == cl0
            m1 = (idx + 1) == cl1
            pos = g * L + lane
            plsc.store_scatter(slab, [cl0, zidx16, pos], low_v, mask=m0)
            plsc.store_scatter(slab, [cl1, zidx16, pos], up_v, mask=m1)
            idxbuf[pl.ds(g * L, L)] = cl0
            idxbuf[pl.ds(C + g * L, L)] = cl1

    def clear_chunk(slab, idxbuf):
        for g in range(GROUPS):
            pos = g * L + lane
            c0 = idxbuf[pl.ds(g * L, L)]
            c1 = idxbuf[pl.ds(C + g * L, L)]
            plsc.store_scatter(slab, [c0, zidx16, pos], zeros16)
            plsc.store_scatter(slab, [c1, zidx16, pos], zeros16)

    def dst_for(k):
        i = row0 + (k >> 4)
        j = col0 + (k & 15) * C
        return out_hbm.at[:, pl.ds(i, 1), pl.ds(j, C)]

    slabs = (slab0, slab1)
    idxs = (idx0, idx1)
    sems = (sem0, sem1)

    for p in range(2):
        fill_chunk(jnp.int32(p), slabs[p], idxs[p])
        pltpu.make_async_copy(slabs[p], dst_for(jnp.int32(p)), sems[p]).start()

    def loop_body(m, _):
        for p in range(2):
            k = 2 * m + p
            pltpu.make_async_copy(slabs[p], dst_for(k - 2), sems[p]).wait()
            clear_chunk(slabs[p], idxs[p])
            fill_chunk(k, slabs[p], idxs[p])
            pltpu.make_async_copy(slabs[p], dst_for(k), sems[p]).start()
        return 0

    lax.fori_loop(1, NCHUNK // 2, loop_body, 0)

    for p in range(2):
        pltpu.make_async_copy(slabs[p], dst_for(jnp.int32(0)), sems[p]).wait()


def kernel(x):
    orig_shape = x.shape[:-1]
    b0, b1 = orig_shape
    xf = x.reshape(b0, b1)
    mesh = plsc.VectorSubcoreMesh(core_axis_name="c", subcore_axis_name="s")
    f = functools.partial(
        pl.kernel,
        mesh=mesh,
        out_type=jax.ShapeDtypeStruct((NUM_BINS, b0, b1), jnp.float32),
        scratch_types=[
            pltpu.VMEM((b0 * b1 // 32,), jnp.float32),
            pltpu.VMEM((NUM_BINS, 1, C), jnp.float32),
            pltpu.VMEM((NUM_BINS, 1, C), jnp.float32),
            pltpu.VMEM((2 * C,), jnp.int32),
            pltpu.VMEM((2 * C,), jnp.int32),
            pltpu.SemaphoreType.DMA,
            pltpu.SemaphoreType.DMA,
        ],
        compiler_params=pltpu.CompilerParams(
            needs_layout_passes=False, use_tc_tiling_on_sc=True
        ),
    )(_sc_body)
    out = f(xf)
    return out.transpose(1, 2, 0)
